# baseline (speedup 1.0000x reference)
import functools

import jax
import jax.numpy as jnp
from jax import lax
from jax.experimental import pallas as pl
from jax.experimental.pallas import tpu as pltpu
from jax.experimental.pallas import tpu_sc as plsc

_VOCAB = 256
_EMB = 32
_B = 4096
_L = 200
_N = _B * _L
_NC = 2
_NS = 16
_NW = _NC * _NS
_N_PER_W = _N // _NW
_CHUNK = 800
_N_CHUNKS = _N_PER_W // _CHUNK
_NSLOT = 4
_SRC_HBM = (False, True, False, True)

_mesh = plsc.VectorSubcoreMesh(core_axis_name="c", subcore_axis_name="s")


@functools.partial(
    pl.kernel,
    out_type=jax.ShapeDtypeStruct((_N, _EMB), jnp.float32),
    mesh=_mesh,
    scratch_types=[
        pltpu.VMEM_SHARED((_VOCAB, _EMB), jnp.float32),
        pltpu.VMEM((_NSLOT, _CHUNK), jnp.int32),
        pltpu.VMEM((_NSLOT, _CHUNK, _EMB), jnp.float32),
        pltpu.SemaphoreType.DMA((_NSLOT,)),
        pltpu.SemaphoreType.DMA((_NSLOT,)),
        pltpu.SemaphoreType.DMA((_NSLOT,)),
    ],
    compiler_params=pltpu.CompilerParams(use_tc_tiling_on_sc=False),
)
def _gather_kernel(ids_hbm, table_hbm, out_hbm, table_s, idx_v, rows_v,
                   sem_idx, sem_g, sem_w):
    wid = lax.axis_index("s") * _NC + lax.axis_index("c")
    base = wid * _N_PER_W

    @pl.when(lax.axis_index("s") == 0)
    def _():
        pltpu.sync_copy(table_hbm, table_s)
    plsc.subcore_barrier()

    def table(slot):
        return table_hbm if _SRC_HBM[slot] else table_s

    for s in range(_NSLOT):
        pltpu.async_copy(ids_hbm.at[pl.ds(base + s * _CHUNK, _CHUNK)],
                         idx_v.at[s], sem_idx.at[s])
    pltpu.make_async_copy(ids_hbm.at[pl.ds(base, _CHUNK)],
                          idx_v.at[0], sem_idx.at[0]).wait()
    pltpu.async_copy(table(0).at[idx_v.at[0]], rows_v.at[0], sem_g.at[0])

    @pl.loop(0, _N_CHUNKS, step=_NSLOT)
    def _steady(i):
        for s in range(_NSLOT):
            s1 = (s + 1) % _NSLOT
            c = i + s
            c1 = c + 1
            off = base + c * _CHUNK

            @pl.when(c1 < _N_CHUNKS)
            def _():
                off1 = base + c1 * _CHUNK
                pltpu.make_async_copy(ids_hbm.at[pl.ds(off1, _CHUNK)],
                                      idx_v.at[s1], sem_idx.at[s1]).wait()

                @pl.when(c1 >= _NSLOT)
                def _():
                    pltpu.make_async_copy(
                        rows_v.at[s1],
                        out_hbm.at[pl.ds(off1 - _NSLOT * _CHUNK, _CHUNK)],
                        sem_w.at[s1]).wait()

                pltpu.async_copy(table(s1).at[idx_v.at[s1]], rows_v.at[s1],
                                 sem_g.at[s1])

            pltpu.make_async_copy(table(s).at[idx_v.at[s]], rows_v.at[s],
                                  sem_g.at[s]).wait()
            pltpu.async_copy(rows_v.at[s], out_hbm.at[pl.ds(off, _CHUNK)],
                             sem_w.at[s])

            @pl.when(c + _NSLOT < _N_CHUNKS)
            def _():
                pltpu.async_copy(
                    ids_hbm.at[pl.ds(off + _NSLOT * _CHUNK, _CHUNK)],
                    idx_v.at[s], sem_idx.at[s])

    for s in range(_NSLOT):
        off = base + (_N_CHUNKS - _NSLOT + s) * _CHUNK
        pltpu.make_async_copy(rows_v.at[s], out_hbm.at[pl.ds(off, _CHUNK)],
                              sem_w.at[s]).wait()


def kernel(char_ids, char_embs):
    ids_flat = char_ids.reshape(_N)
    out = _gather_kernel(ids_flat, char_embs)
    return out.reshape(_B, _L, _EMB)


# speedup vs baseline: 1.2700x; 1.2700x over previous
import functools

import jax
import jax.numpy as jnp
from jax import lax
from jax.experimental import pallas as pl
from jax.experimental.pallas import tpu as pltpu
from jax.experimental.pallas import tpu_sc as plsc

_VOCAB = 256
_EMB = 32
_B = 4096
_L = 200
_N = _B * _L
_NC = 2
_NS = 16
_NW = _NC * _NS
_N_PER_W = _N // _NW
_CHUNK = 800
_N_CHUNKS = _N_PER_W // _CHUNK
_NSLOT = 4
_SRC_HBM = (False, False, False, False)

_mesh = plsc.VectorSubcoreMesh(core_axis_name="c", subcore_axis_name="s")


@functools.partial(
    pl.kernel,
    out_type=jax.ShapeDtypeStruct((_N, _EMB), jnp.float32),
    mesh=_mesh,
    scratch_types=[
        pltpu.VMEM_SHARED((_VOCAB, _EMB), jnp.float32),
        pltpu.VMEM((_NSLOT, _CHUNK), jnp.int32),
        pltpu.VMEM((_NSLOT, _CHUNK, _EMB), jnp.float32),
        pltpu.SemaphoreType.DMA((_NSLOT,)),
        pltpu.SemaphoreType.DMA((_NSLOT,)),
        pltpu.SemaphoreType.DMA((_NSLOT,)),
    ],
    compiler_params=pltpu.CompilerParams(use_tc_tiling_on_sc=False),
)
def _gather_kernel(ids_hbm, table_hbm, out_hbm, table_s, idx_v, rows_v,
                   sem_idx, sem_g, sem_w):
    wid = lax.axis_index("s") * _NC + lax.axis_index("c")
    base = wid * _N_PER_W

    @pl.when(lax.axis_index("s") == 0)
    def _():
        pltpu.sync_copy(table_hbm, table_s)
    plsc.subcore_barrier()

    def table(slot):
        return table_hbm if _SRC_HBM[slot] else table_s

    for s in range(_NSLOT):
        pltpu.async_copy(ids_hbm.at[pl.ds(base + s * _CHUNK, _CHUNK)],
                         idx_v.at[s], sem_idx.at[s])
    pltpu.make_async_copy(ids_hbm.at[pl.ds(base, _CHUNK)],
                          idx_v.at[0], sem_idx.at[0]).wait()
    pltpu.async_copy(table(0).at[idx_v.at[0]], rows_v.at[0], sem_g.at[0])

    @pl.loop(0, _N_CHUNKS, step=_NSLOT)
    def _steady(i):
        for s in range(_NSLOT):
            s1 = (s + 1) % _NSLOT
            c = i + s
            c1 = c + 1
            off = base + c * _CHUNK

            @pl.when(c1 < _N_CHUNKS)
            def _():
                off1 = base + c1 * _CHUNK
                pltpu.make_async_copy(ids_hbm.at[pl.ds(off1, _CHUNK)],
                                      idx_v.at[s1], sem_idx.at[s1]).wait()

                @pl.when(c1 >= _NSLOT)
                def _():
                    pltpu.make_async_copy(
                        rows_v.at[s1],
                        out_hbm.at[pl.ds(off1 - _NSLOT * _CHUNK, _CHUNK)],
                        sem_w.at[s1]).wait()

                pltpu.async_copy(table(s1).at[idx_v.at[s1]], rows_v.at[s1],
                                 sem_g.at[s1])

            pltpu.make_async_copy(table(s).at[idx_v.at[s]], rows_v.at[s],
                                  sem_g.at[s]).wait()
            pltpu.async_copy(rows_v.at[s], out_hbm.at[pl.ds(off, _CHUNK)],
                             sem_w.at[s])

            @pl.when(c + _NSLOT < _N_CHUNKS)
            def _():
                pltpu.async_copy(
                    ids_hbm.at[pl.ds(off + _NSLOT * _CHUNK, _CHUNK)],
                    idx_v.at[s], sem_idx.at[s])

    for s in range(_NSLOT):
        off = base + (_N_CHUNKS - _NSLOT + s) * _CHUNK
        pltpu.make_async_copy(rows_v.at[s], out_hbm.at[pl.ds(off, _CHUNK)],
                              sem_w.at[s]).wait()


def kernel(char_ids, char_embs):
    ids_flat = char_ids.reshape(_N)
    out = _gather_kernel(ids_flat, char_embs)
    return out.reshape(_B, _L, _EMB)
